# Initial kernel scaffold; baseline (speedup 1.0000x reference)
import jax
import jax.numpy as jnp
from jax.experimental import pallas as pl


def kernel(syndrome, stab_id, cycle_id, stab_emb, cycle_emb, val_emb):
    raise NotImplementedError("write your pallas kernel here")


# speedup vs baseline: 18.1943x; 18.1943x over previous
import functools

import jax
import jax.numpy as jnp
from jax import lax
from jax.experimental import pallas as pl
from jax.experimental.pallas import tpu as pltpu
from jax.experimental.pallas import tpu_sc as plsc


def _sc_base_tables(stab_id, cycle_id, stab_emb, cycle_emb, val_emb):
    (L,) = stab_id.shape
    D = stab_emb.shape[1]
    info = plsc.get_sparse_core_info()
    NC, NS = info.num_cores, info.num_subcores
    NW = NC * NS
    RPW = 8
    n_active = L // RPW
    assert n_active * RPW == L and n_active <= NW

    mesh = plsc.VectorSubcoreMesh(core_axis_name="c", subcore_axis_name="s")

    @functools.partial(
        pl.kernel,
        out_type=(
            jax.ShapeDtypeStruct((L, D), jnp.float32),
            jax.ShapeDtypeStruct((L, D), jnp.float32),
        ),
        mesh=mesh,
        scratch_types=[
            pltpu.VMEM((16,), jnp.int32),
            pltpu.VMEM((16,), jnp.int32),
            pltpu.VMEM((RPW, D), jnp.float32),
            pltpu.VMEM((RPW, D), jnp.float32),
            pltpu.VMEM((2, D), jnp.float32),
            pltpu.VMEM((RPW, D), jnp.float32),
            pltpu.VMEM((RPW, D), jnp.float32),
            pltpu.SemaphoreType.DMA,
            pltpu.SemaphoreType.DMA,
        ],
    )
    def sc_kernel(stab_id_hbm, cycle_id_hbm, stab_emb_hbm, cycle_emb_hbm,
                  val_emb_hbm, out0_hbm, out1_hbm,
                  sidx_v, cidx_v, srows_v, crows_v, val_v, o0_v, o1_v,
                  sem0, sem1):
        wid = lax.axis_index("s") * NC + lax.axis_index("c")

        @pl.when(wid < n_active)
        def _():
            base = wid * RPW
            pltpu.sync_copy(stab_id_hbm.at[pl.ds(base, RPW)],
                            sidx_v.at[pl.ds(0, RPW)])
            pltpu.sync_copy(cycle_id_hbm.at[pl.ds(base, RPW)],
                            cidx_v.at[pl.ds(0, RPW)])
            pltpu.sync_copy(val_emb_hbm, val_v)
            siv = sidx_v[...]
            civ = cidx_v[...]
            for r in range(RPW):
                pltpu.async_copy(stab_emb_hbm.at[pl.ds(siv[r], 1)],
                                 srows_v.at[pl.ds(r, 1)], sem0)
                pltpu.async_copy(cycle_emb_hbm.at[pl.ds(civ[r], 1)],
                                 crows_v.at[pl.ds(r, 1)], sem1)
            for r in range(RPW):
                pltpu.make_async_copy(stab_emb_hbm.at[pl.ds(0, 1)],
                                      srows_v.at[pl.ds(r, 1)], sem0).wait()
                pltpu.make_async_copy(cycle_emb_hbm.at[pl.ds(0, 1)],
                                      crows_v.at[pl.ds(r, 1)], sem1).wait()
            for r in range(RPW):
                for c in range(D // 16):
                    sl = pl.ds(c * 16, 16)
                    e = srows_v[r, sl] + crows_v[r, sl]
                    o0_v[r, sl] = e + val_v[0, sl]
                    o1_v[r, sl] = e + val_v[1, sl]
            pltpu.sync_copy(o0_v, out0_hbm.at[pl.ds(base, RPW)])
            pltpu.sync_copy(o1_v, out1_hbm.at[pl.ds(base, RPW)])

    return sc_kernel(stab_id, cycle_id, stab_emb, cycle_emb, val_emb)


def _tc_expand(syndrome_t, base0, base1):
    L, B = syndrome_t.shape
    D = base0.shape[1]
    TB = 128
    assert B % TB == 0

    def body(syn_ref, b0_ref, b1_ref, out_ref):
        b0 = b0_ref[...]
        dvec = b1_ref[...] - b0
        for t in range(TB):
            synf = syn_ref[:, pl.ds(t, 1)].astype(jnp.float32)
            sel = jnp.broadcast_to(synf, (L, D))
            out_ref[pl.ds(t, 1)] = (b0 + sel * dvec)[None]

    return pl.pallas_call(
        body,
        grid=(B // TB,),
        in_specs=[
            pl.BlockSpec((L, TB), lambda i: (0, i)),
            pl.BlockSpec((L, D), lambda i: (0, 0)),
            pl.BlockSpec((L, D), lambda i: (0, 0)),
        ],
        out_specs=pl.BlockSpec((TB, L, D), lambda i: (i, 0, 0)),
        out_shape=jax.ShapeDtypeStruct((B, L, D), jnp.float32),
    )(syndrome_t, base0, base1)


def kernel(syndrome, stab_id, cycle_id, stab_emb, cycle_emb, val_emb):
    syndrome = syndrome.astype(jnp.int32)
    stab_id = stab_id.astype(jnp.int32)
    cycle_id = cycle_id.astype(jnp.int32)
    base0, base1 = _sc_base_tables(stab_id, cycle_id, stab_emb, cycle_emb,
                                   val_emb)
    return _tc_expand(syndrome.T, base0, base1)
